# baseline (speedup 1.0000x reference)
import jax
import jax.numpy as jnp
from jax import lax
from jax.experimental import pallas as pl
from jax.experimental.pallas import tpu as pltpu
from jax.experimental.pallas import tpu_sc as plsc

SX = SY = SZ = 64
ROW = SY * SX
B = 16
N = SX * SY * SZ
NC, NS, L = 2, 16, 16
NQUAD = 8
TR, TC_ = B // 8, N // 128
ZC = ROW // 128


def _gridding_reverse_sc():
    mesh = plsc.VectorSubcoreMesh(
        core_axis_name="c", subcore_axis_name="s", num_cores=NC, num_subcores=NS
    )

    slab_t = pltpu.VMEM((ZC, 128), jnp.float32)

    @pl.kernel(
        out_type=jax.ShapeDtypeStruct((3, TR, TC_, 8, 128), jnp.float32),
        mesh=mesh,
        compiler_params=pltpu.CompilerParams(
            needs_layout_passes=False, use_tc_tiling_on_sc=False
        ),
        scratch_types=(
            [slab_t] * 4
            + [slab_t] * 4
            + [slab_t] * 6
            + [slab_t] * 6
            + [
                pltpu.VMEM((3 * L,), jnp.float32),
                pltpu.SemaphoreType.DMA,
                pltpu.SemaphoreType.DMA,
                pltpu.SemaphoreType.DMA,
            ]
        ),
    )
    def k(grid_hbm, osf_hbm, out_hbm, w0, w1, w2, w3, s0b, s1b, s2b, s3b,
          axA, ayA, azA, bxA, byA, bzA, axB, ayB, azB, bxB, byB, bzB,
          osf_v, semIn, semA, semB):
        wid = lax.axis_index("s") * NC + lax.axis_index("c")
        b = wid >> 1
        odd = wid & 1
        zstart = odd * (SZ // 2)
        tr = b >> 3
        sl = b & 7

        pltpu.sync_copy(osf_hbm, osf_v)
        sc0 = osf_v[pl.ds(0, L)]
        sc1 = osf_v[pl.ds(L, L)]
        sc2 = osf_v[pl.ds(2 * L, L)]

        iota = lax.iota(jnp.int32, L)
        zeros = jnp.zeros((L,), jnp.float32)

        def in_slice(z):
            return grid_hbm.at[tr, pl.ds(z * ZC, ZC), sl]

        def out_slice(cc, z):
            return out_hbm.at[cc, tr, pl.ds(z * ZC, ZC), sl]

        def pass1(gbuf, sxbuf):
            @plsc.parallel_loop(0, ROW, L, unroll=8)
            def _(c):
                g0v = gbuf[c >> 7, pl.ds(c & 127, L)]
                f = jnp.maximum(iota + (c - 1), 0)
                gm = plsc.load_gather(gbuf, [f >> 7, f & 127])
                sxbuf[c >> 7, pl.ds(c & 127, L)] = g0v + gm

        def pass2ab(gP, sP, gM, sM, gN, sN, pxa, pya, pza, pxb, pyb, pzb, z):
            zfA = z.astype(jnp.float32) - 32.0
            zfB = zfA + 1.0

            for kk in range(4):
                pxa[0, pl.ds(kk * L, L)] = zeros
                pya[0, pl.ds(kk * L, L)] = zeros
                pza[0, pl.ds(kk * L, L)] = zeros
                pxb[0, pl.ds(kk * L, L)] = zeros
                pyb[0, pl.ds(kk * L, L)] = zeros
                pzb[0, pl.ds(kk * L, L)] = zeros

            @plsc.parallel_loop(64, ROW, L, unroll=2)
            def _(c):
                r0 = c >> 7
                c0 = c & 127
                d = c - 64
                r1 = d >> 7
                c1 = d & 127
                sP0 = sP[r0, pl.ds(c0, L)]
                sP1 = sP[r1, pl.ds(c1, L)]
                sM0 = sM[r0, pl.ds(c0, L)]
                sM1 = sM[r1, pl.ds(c1, L)]
                sN0 = sN[r0, pl.ds(c0, L)]
                sN1 = sN[r1, pl.ds(c1, L)]
                gP0 = gP[r0, pl.ds(c0, L)]
                gP1 = gP[r1, pl.ds(c1, L)]
                gM0 = gM[r0, pl.ds(c0, L)]
                gM1 = gM[r1, pl.ds(c1, L)]
                gN0 = gN[r0, pl.ds(c0, L)]
                gN1 = gN[r1, pl.ds(c1, L)]
                mP = sP0 + sP1
                mM = sM0 + sM1
                mN = sN0 + sN1
                wA = mM + mP
                wB = mN + mM
                gPs = gP0 + gP1
                gMs = gM0 + gM1
                gNs = gN0 + gN1
                sx1A = wA - (gPs + gMs)
                sx1B = wB - (gMs + gNs)
                sy1A = sM1 + sP1
                sy1B = sN1 + sM1
                rA = 1.0 / wA
                rB = 1.0 / wB
                xi = iota + (c & (SX - 1))
                y = c >> 6
                yf = y.astype(jnp.float32) - 32.0
                xf = xi.astype(jnp.float32) - 32.0
                mgeo = xi > 0
                mA = (wA > 0.0) & mgeo
                mB = (wB > 0.0) & mgeo
                pxa[r0, pl.ds(c0, L)] = jnp.where(mA, (xf - sx1A * rA) * sc0, 0.0)
                pya[r0, pl.ds(c0, L)] = jnp.where(mA, (yf - sy1A * rA) * sc1, 0.0)
                pza[r0, pl.ds(c0, L)] = jnp.where(mA, (zfA - mP * rA) * sc2, 0.0)
                pxb[r0, pl.ds(c0, L)] = jnp.where(mB, (xf - sx1B * rB) * sc0, 0.0)
                pyb[r0, pl.ds(c0, L)] = jnp.where(mB, (yf - sy1B * rB) * sc1, 0.0)
                pzb[r0, pl.ds(c0, L)] = jnp.where(mB, (zfB - mM * rB) * sc2, 0.0)

        def out_start6(px1, py1, pz1, px2, py2, pz2, z, sem):
            pltpu.async_copy(px1, out_slice(0, z), sem)
            pltpu.async_copy(py1, out_slice(1, z), sem)
            pltpu.async_copy(pz1, out_slice(2, z), sem)
            pltpu.async_copy(px2, out_slice(0, z + 1), sem)
            pltpu.async_copy(py2, out_slice(1, z + 1), sem)
            pltpu.async_copy(pz2, out_slice(2, z + 1), sem)

        def out_drain6(bufs, sem):
            for i, bb in enumerate(bufs):
                pltpu.make_async_copy(bb, out_slice(i % 3, 0), sem).wait()

        def in_wait(z, dst):
            pltpu.make_async_copy(in_slice(z), dst, semIn).wait()

        z_prev0 = jnp.maximum(zstart - 1, 0)
        pltpu.sync_copy(in_slice(z_prev0), w0)
        pass1(w0, s0b)
        pltpu.async_copy(in_slice(zstart), w1, semIn)
        pltpu.async_copy(in_slice(zstart + 1), w2, semIn)

        setA = (axA, ayA, azA, bxA, byA, bzA)
        setB = (axB, ayB, azB, bxB, byB, bzB)

        def quad(q, _):
            zA = zstart + 4 * q

            in_wait(zA, w1)
            in_wait(zA + 1, w2)
            pass1(w1, s1b)
            pass1(w2, s2b)

            @pl.when(q > 0)
            def _():
                out_drain6(setA, semA)

            pass2ab(w0, s0b, w1, s1b, w2, s2b, *setA, zA)
            out_start6(*setA, zA, semA)
            pltpu.async_copy(in_slice(zA + 2), w3, semIn)
            pltpu.async_copy(in_slice(zA + 3), w0, semIn)

            in_wait(zA + 2, w3)
            in_wait(zA + 3, w0)
            pass1(w3, s3b)
            pass1(w0, s0b)

            @pl.when(q > 0)
            def _():
                out_drain6(setB, semB)

            pass2ab(w2, s2b, w3, s3b, w0, s0b, *setB, zA + 2)
            out_start6(*setB, zA + 2, semB)

            @pl.when(q < NQUAD - 1)
            def _():
                pltpu.async_copy(in_slice(zA + 4), w1, semIn)
                pltpu.async_copy(in_slice(zA + 5), w2, semIn)

            return 0

        lax.fori_loop(0, NQUAD, quad, 0)
        out_drain6(setA, semA)
        out_drain6(setB, semB)

        @pl.when(odd == 0)
        def _():
            @plsc.parallel_loop(0, ROW, L)
            def _(c):
                axA[c >> 7, pl.ds(c & 127, L)] = zeros
                ayA[c >> 7, pl.ds(c & 127, L)] = zeros
                azA[c >> 7, pl.ds(c & 127, L)] = zeros

            pltpu.sync_copy(axA, out_slice(0, 0))
            pltpu.sync_copy(ayA, out_slice(1, 0))
            pltpu.sync_copy(azA, out_slice(2, 0))

    return k


def kernel(grid, output_scaling_factors):
    osf_exp = jnp.repeat(output_scaling_factors, L)
    grid4 = grid.reshape(TR, 8, TC_, 128).transpose(0, 2, 1, 3)
    out5 = _gridding_reverse_sc()(grid4, osf_exp)
    out = out5.transpose(0, 1, 3, 2, 4).reshape(3, B, N)
    return out.transpose(1, 2, 0)


# speedup vs baseline: 1.1065x; 1.1065x over previous
import jax
import jax.numpy as jnp
from jax import lax
from jax.experimental import pallas as pl
from jax.experimental.pallas import tpu as pltpu
from jax.experimental.pallas import tpu_sc as plsc

SX = SY = SZ = 64
ROW = SY * SX
B = 16
N = SX * SY * SZ
NC, NS, L = 2, 16, 16
NQUAD = 8
TR, TC_ = B // 8, N // 128
ZC = ROW // 128


def _gridding_reverse_sc():
    mesh = plsc.VectorSubcoreMesh(
        core_axis_name="c", subcore_axis_name="s", num_cores=NC, num_subcores=NS
    )

    slab_t = pltpu.VMEM((ZC, 128), jnp.float32)

    @pl.kernel(
        out_type=jax.ShapeDtypeStruct((3, TR, TC_, 8, 128), jnp.float32),
        mesh=mesh,
        compiler_params=pltpu.CompilerParams(
            needs_layout_passes=False, use_tc_tiling_on_sc=False
        ),
        scratch_types=(
            [slab_t] * 4
            + [slab_t] * 4
            + [slab_t] * 6
            + [slab_t] * 6
            + [
                pltpu.VMEM((3 * L,), jnp.float32),
                pltpu.SemaphoreType.DMA,
                pltpu.SemaphoreType.DMA,
                pltpu.SemaphoreType.DMA,
                pltpu.SemaphoreType.DMA,
                pltpu.SemaphoreType.DMA,
                pltpu.SemaphoreType.DMA,
            ]
        ),
    )
    def k(grid_hbm, osf_hbm, out_hbm, w0, w1, w2, w3, s0b, s1b, s2b, s3b,
          axA, ayA, azA, bxA, byA, bzA, axB, ayB, azB, bxB, byB, bzB,
          osf_v, semI0, semI1, semI2, semI3, semA, semB):
        wid = lax.axis_index("s") * NC + lax.axis_index("c")
        b = wid >> 1
        odd = wid & 1
        zstart = odd * (SZ // 2)
        tr = b >> 3
        sl = b & 7

        pltpu.sync_copy(osf_hbm, osf_v)
        sc0 = osf_v[pl.ds(0, L)]
        sc1 = osf_v[pl.ds(L, L)]
        sc2 = osf_v[pl.ds(2 * L, L)]

        iota = lax.iota(jnp.int32, L)
        zeros = jnp.zeros((L,), jnp.float32)

        def in_slice(z):
            return grid_hbm.at[tr, pl.ds(z * ZC, ZC), sl]

        def out_slice(cc, z):
            return out_hbm.at[cc, tr, pl.ds(z * ZC, ZC), sl]

        def pass1(gbuf, sxbuf):
            @plsc.parallel_loop(0, ROW, L, unroll=8)
            def _(c):
                g0v = gbuf[c >> 7, pl.ds(c & 127, L)]
                f = jnp.maximum(iota + (c - 1), 0)
                gm = plsc.load_gather(gbuf, [f >> 7, f & 127])
                sxbuf[c >> 7, pl.ds(c & 127, L)] = g0v + gm

        def pass2ab(gP, sP, gM, sM, gN, sN, pxa, pya, pza, pxb, pyb, pzb, z):
            zfA = z.astype(jnp.float32) - 32.0
            zfB = zfA + 1.0

            for kk in range(4):
                pxa[0, pl.ds(kk * L, L)] = zeros
                pya[0, pl.ds(kk * L, L)] = zeros
                pza[0, pl.ds(kk * L, L)] = zeros
                pxb[0, pl.ds(kk * L, L)] = zeros
                pyb[0, pl.ds(kk * L, L)] = zeros
                pzb[0, pl.ds(kk * L, L)] = zeros

            @plsc.parallel_loop(64, ROW, L, unroll=4)
            def _(c):
                r0 = c >> 7
                c0 = c & 127
                d = c - 64
                r1 = d >> 7
                c1 = d & 127
                sP0 = sP[r0, pl.ds(c0, L)]
                sP1 = sP[r1, pl.ds(c1, L)]
                sM0 = sM[r0, pl.ds(c0, L)]
                sM1 = sM[r1, pl.ds(c1, L)]
                sN0 = sN[r0, pl.ds(c0, L)]
                sN1 = sN[r1, pl.ds(c1, L)]
                gP0 = gP[r0, pl.ds(c0, L)]
                gP1 = gP[r1, pl.ds(c1, L)]
                gM0 = gM[r0, pl.ds(c0, L)]
                gM1 = gM[r1, pl.ds(c1, L)]
                gN0 = gN[r0, pl.ds(c0, L)]
                gN1 = gN[r1, pl.ds(c1, L)]
                mP = sP0 + sP1
                mM = sM0 + sM1
                mN = sN0 + sN1
                wA = mM + mP
                wB = mN + mM
                gPs = gP0 + gP1
                gMs = gM0 + gM1
                gNs = gN0 + gN1
                sx1A = wA - (gPs + gMs)
                sx1B = wB - (gMs + gNs)
                sy1A = sM1 + sP1
                sy1B = sN1 + sM1
                rA = 1.0 / wA
                rB = 1.0 / wB
                xi = iota + (c & (SX - 1))
                y = c >> 6
                yf = y.astype(jnp.float32) - 32.0
                xf = xi.astype(jnp.float32) - 32.0
                mgeo = xi > 0
                mA = (wA > 0.0) & mgeo
                mB = (wB > 0.0) & mgeo
                pxa[r0, pl.ds(c0, L)] = jnp.where(mA, (xf - sx1A * rA) * sc0, 0.0)
                pya[r0, pl.ds(c0, L)] = jnp.where(mA, (yf - sy1A * rA) * sc1, 0.0)
                pza[r0, pl.ds(c0, L)] = jnp.where(mA, (zfA - mP * rA) * sc2, 0.0)
                pxb[r0, pl.ds(c0, L)] = jnp.where(mB, (xf - sx1B * rB) * sc0, 0.0)
                pyb[r0, pl.ds(c0, L)] = jnp.where(mB, (yf - sy1B * rB) * sc1, 0.0)
                pzb[r0, pl.ds(c0, L)] = jnp.where(mB, (zfB - mM * rB) * sc2, 0.0)

        def out_start6(px1, py1, pz1, px2, py2, pz2, z, sem):
            pltpu.async_copy(px1, out_slice(0, z), sem)
            pltpu.async_copy(py1, out_slice(1, z), sem)
            pltpu.async_copy(pz1, out_slice(2, z), sem)
            pltpu.async_copy(px2, out_slice(0, z + 1), sem)
            pltpu.async_copy(py2, out_slice(1, z + 1), sem)
            pltpu.async_copy(pz2, out_slice(2, z + 1), sem)

        def out_drain6(bufs, sem):
            for i, bb in enumerate(bufs):
                pltpu.make_async_copy(bb, out_slice(i % 3, 0), sem).wait()

        def in_wait(z, dst, sem):
            pltpu.make_async_copy(in_slice(z), dst, sem).wait()

        z_prev0 = jnp.maximum(zstart - 1, 0)
        pltpu.sync_copy(in_slice(z_prev0), w0)
        pass1(w0, s0b)
        pltpu.async_copy(in_slice(zstart), w1, semI1)
        pltpu.async_copy(in_slice(zstart + 1), w2, semI2)

        setA = (axA, ayA, azA, bxA, byA, bzA)
        setB = (axB, ayB, azB, bxB, byB, bzB)

        def quad(q, _):
            zA = zstart + 4 * q

            pltpu.async_copy(in_slice(zA + 2), w3, semI3)

            in_wait(zA, w1, semI1)
            in_wait(zA + 1, w2, semI2)
            pass1(w1, s1b)
            pass1(w2, s2b)

            @pl.when(q > 0)
            def _():
                out_drain6(setA, semA)

            pass2ab(w0, s0b, w1, s1b, w2, s2b, *setA, zA)
            pltpu.async_copy(in_slice(zA + 3), w0, semI0)

            @pl.when(q < NQUAD - 1)
            def _():
                pltpu.async_copy(in_slice(zA + 4), w1, semI1)

            out_start6(*setA, zA, semA)

            in_wait(zA + 2, w3, semI3)
            pass1(w3, s3b)
            in_wait(zA + 3, w0, semI0)
            pass1(w0, s0b)

            @pl.when(q > 0)
            def _():
                out_drain6(setB, semB)

            pass2ab(w2, s2b, w3, s3b, w0, s0b, *setB, zA + 2)
            out_start6(*setB, zA + 2, semB)

            @pl.when(q < NQUAD - 1)
            def _():
                pltpu.async_copy(in_slice(zA + 5), w2, semI2)

            return 0

        lax.fori_loop(0, NQUAD, quad, 0)
        out_drain6(setA, semA)
        out_drain6(setB, semB)

        @pl.when(odd == 0)
        def _():
            @plsc.parallel_loop(0, ROW, L)
            def _(c):
                axA[c >> 7, pl.ds(c & 127, L)] = zeros
                ayA[c >> 7, pl.ds(c & 127, L)] = zeros
                azA[c >> 7, pl.ds(c & 127, L)] = zeros

            pltpu.sync_copy(axA, out_slice(0, 0))
            pltpu.sync_copy(ayA, out_slice(1, 0))
            pltpu.sync_copy(azA, out_slice(2, 0))

    return k


def kernel(grid, output_scaling_factors):
    osf_exp = jnp.repeat(output_scaling_factors, L)
    grid4 = grid.reshape(TR, 8, TC_, 128).transpose(0, 2, 1, 3)
    out5 = _gridding_reverse_sc()(grid4, osf_exp)
    out = out5.transpose(0, 1, 3, 2, 4).reshape(3, B, N)
    return out.transpose(1, 2, 0)
